# Initial kernel scaffold; baseline (speedup 1.0000x reference)
import jax
import jax.numpy as jnp
from jax.experimental import pallas as pl


def kernel(xKeyValue, xQuery, W_val, b_val, W_key, b_key, W_q, b_q, W_proj, b_proj):
    raise NotImplementedError("write your pallas kernel here")


# speedup vs baseline: 1.3413x; 1.3413x over previous
import functools

import jax
import jax.numpy as jnp
from jax import lax
from jax.experimental import pallas as pl
from jax.experimental.pallas import tpu as pltpu

B, C, H, W = 2, 96, 384, 384
NPIX = H * W
FAC = 6
HC = H // FAC
NCOARSE = HC * HC
SPLITM = 32
NBLK = SPLITM * SPLITM
SFINE = 12
NQ = SFINE * SFINE
TILE = FAC * FAC
INV_SC2 = 1.0 / (96.0 ** 0.5)


def _mm_body(a_ref, b_ref, o_ref):
    o_ref[...] = jnp.dot(a_ref[...], b_ref[...],
                         preferred_element_type=jnp.float32, precision=lax.Precision.HIGHEST)


def _mm(a, bmat, rb):
    m, k = a.shape
    _, n = bmat.shape
    return pl.pallas_call(
        _mm_body,
        grid=(m // rb,),
        in_specs=[pl.BlockSpec((rb, k), lambda i: (i, 0)),
                  pl.BlockSpec((k, n), lambda i: (0, 0))],
        out_specs=pl.BlockSpec((rb, n), lambda i: (i, 0)),
        out_shape=jax.ShapeDtypeStruct((m, n), jnp.float32),
    )(a, bmat)


def _kv_proj_body(x_ref, wt_ref, bk_ref, bv_ref, key_ref, val_ref):
    d = jnp.dot(x_ref[0], wt_ref[...], preferred_element_type=jnp.float32, precision=lax.Precision.HIGHEST)
    key_ref[0] = d[:, 0:C] + bk_ref[...]
    val_ref[0] = d[:, C:2 * C] + bv_ref[...]


def _q_proj_body(x_ref, wt_ref, b_ref, o_ref):
    o_ref[0] = jnp.dot(x_ref[0], wt_ref[...],
                       preferred_element_type=jnp.float32, precision=lax.Precision.HIGHEST) + b_ref[...]


def _cproj_body(pq_ref, wqt_ref, bq_ref, pk_ref, wkt_ref, bk_ref,
                qd_ref, kd_ref):
    qd_ref[0] = jnp.dot(pq_ref[0], wqt_ref[...],
                        preferred_element_type=jnp.float32, precision=lax.Precision.HIGHEST) + bq_ref[...]
    kd_ref[0] = jnp.dot(pk_ref[0], wkt_ref[...],
                        preferred_element_type=jnp.float32, precision=lax.Precision.HIGHEST) + bk_ref[...]


def _topk_body(qd_ref, kd_ref, pair_ref, o_ref):
    q = qd_ref[0]
    k = kd_ref[0]
    e = (2.0 * lax.dot_general(q, k, (((1,), (1,)), ((), ())),
                               preferred_element_type=jnp.float32, precision=lax.Precision.HIGHEST)
         - jnp.sum(k * k, axis=-1)[None, :]) * INV_SC2
    m = jnp.max(e, axis=-1, keepdims=True)
    p = jnp.exp(e - m)
    a = p / jnp.sum(p, axis=-1, keepdims=True)
    srow = a[0:HC, :] + a[HC:2 * HC, :]
    s = jnp.dot(pair_ref[...], srow, preferred_element_type=jnp.float32, precision=lax.Precision.HIGHEST)
    idx = lax.broadcasted_iota(jnp.int32, s.shape, 1)
    m1 = jnp.max(s, axis=-1, keepdims=True)
    i1 = jnp.min(jnp.where(s == m1, idx, NCOARSE), axis=-1, keepdims=True)
    s2 = jnp.where(idx == i1, -jnp.inf, s)
    m2 = jnp.max(s2, axis=-1, keepdims=True)
    i2 = jnp.min(jnp.where(s2 == m2, idx, NCOARSE), axis=-1, keepdims=True)
    o_ref[0] = jnp.concatenate([i1, i2], axis=-1)


def _phase2_body(tk_ref, q_ref, k0_ref, k1_ref, v0_ref, v1_ref, o_ref):
    del tk_ref
    q = q_ref[0, 0]
    k0 = k0_ref[0, 0]
    k1 = k1_ref[0, 0]
    e0 = (2.0 * lax.dot_general(q, k0, (((1,), (1,)), ((), ())),
                                preferred_element_type=jnp.float32, precision=lax.Precision.HIGHEST)
          - jnp.sum(k0 * k0, axis=-1)[None, :]) * INV_SC2
    e1 = (2.0 * lax.dot_general(q, k1, (((1,), (1,)), ((), ())),
                                preferred_element_type=jnp.float32, precision=lax.Precision.HIGHEST)
          - jnp.sum(k1 * k1, axis=-1)[None, :]) * INV_SC2
    m = jnp.maximum(jnp.max(e0, axis=-1, keepdims=True),
                    jnp.max(e1, axis=-1, keepdims=True))
    p0 = jnp.exp(e0 - m)
    p1 = jnp.exp(e1 - m)
    z = jnp.sum(p0, axis=-1, keepdims=True) + jnp.sum(p1, axis=-1, keepdims=True)
    o_ref[0, 0] = (
        jnp.dot(p0 / z, v0_ref[0, 0], preferred_element_type=jnp.float32, precision=lax.Precision.HIGHEST)
        + jnp.dot(p1 / z, v1_ref[0, 0], preferred_element_type=jnp.float32, precision=lax.Precision.HIGHEST))


def _final_proj_body(xq_ref, at_ref, wp1_ref, wp2_ref, bp_ref, o_ref):
    o_ref[0] = (lax.dot_general(wp1_ref[...], xq_ref[0], (((1,), (1,)), ((), ())),
                                preferred_element_type=jnp.float32, precision=lax.Precision.HIGHEST)
                + lax.dot_general(wp2_ref[...], at_ref[0], (((1,), (1,)), ((), ())),
                                  preferred_element_type=jnp.float32, precision=lax.Precision.HIGHEST)
                + bp_ref[...])


def _pool_img(x, pt):
    a = x.reshape(B * C * H, W)
    c1 = _mm(a, pt, 1024)
    c1 = c1.reshape(B, C, H, HC).transpose(0, 1, 3, 2).reshape(B * C * HC, H)
    c2 = _mm(c1, pt, 1024)
    return c2.reshape(B, C, HC, HC).transpose(0, 3, 2, 1).reshape(B, NCOARSE, C)


def kernel(xKeyValue, xQuery, W_val, b_val, W_key, b_key, W_q, b_q,
           W_proj, b_proj):
    f32 = jnp.float32
    pt = (jnp.arange(W)[:, None] // FAC == jnp.arange(HC)[None, :]).astype(f32) / FAC
    pair = (jnp.arange(SPLITM)[:, None] == jnp.arange(HC)[None, :] // 2).astype(f32)

    xkv_r = (xKeyValue.reshape(B, C, HC, FAC, HC, FAC)
             .transpose(0, 2, 4, 3, 5, 1).reshape(B, NPIX, C))
    xq_r = (xQuery.reshape(B, C, SPLITM, SFINE, SPLITM, SFINE)
            .transpose(0, 2, 4, 3, 5, 1).reshape(B, NPIX, C))

    rb = 4608
    grid_r = (B, NPIX // rb)
    keyr, valr = pl.pallas_call(
        _kv_proj_body,
        grid=grid_r,
        in_specs=[pl.BlockSpec((1, rb, C), lambda b, j: (b, j, 0)),
                  pl.BlockSpec((C, 2 * C), lambda b, j: (0, 0)),
                  pl.BlockSpec((1, C), lambda b, j: (0, 0)),
                  pl.BlockSpec((1, C), lambda b, j: (0, 0))],
        out_specs=[pl.BlockSpec((1, rb, C), lambda b, j: (b, j, 0)),
                   pl.BlockSpec((1, rb, C), lambda b, j: (b, j, 0))],
        out_shape=[jax.ShapeDtypeStruct((B, NPIX, C), f32),
                   jax.ShapeDtypeStruct((B, NPIX, C), f32)],
    )(xkv_r, jnp.concatenate([W_key.T, W_val.T], axis=1),
      b_key[None, :], b_val[None, :])

    qt = pl.pallas_call(
        _q_proj_body,
        grid=grid_r,
        in_specs=[pl.BlockSpec((1, rb, C), lambda b, j: (b, j, 0)),
                  pl.BlockSpec((C, C), lambda b, j: (0, 0)),
                  pl.BlockSpec((1, C), lambda b, j: (0, 0))],
        out_specs=pl.BlockSpec((1, rb, C), lambda b, j: (b, j, 0)),
        out_shape=jax.ShapeDtypeStruct((B, NPIX, C), f32),
    )(xq_r, W_q.T, b_q[None, :])

    pq_t = _pool_img(xQuery, pt)
    pkv_t = _pool_img(xKeyValue, pt)

    qd, kd = pl.pallas_call(
        _cproj_body,
        grid=(B,),
        in_specs=[pl.BlockSpec((1, NCOARSE, C), lambda b: (b, 0, 0)),
                  pl.BlockSpec((C, C), lambda b: (0, 0)),
                  pl.BlockSpec((1, C), lambda b: (0, 0)),
                  pl.BlockSpec((1, NCOARSE, C), lambda b: (b, 0, 0)),
                  pl.BlockSpec((C, C), lambda b: (0, 0)),
                  pl.BlockSpec((1, C), lambda b: (0, 0))],
        out_specs=[pl.BlockSpec((1, NCOARSE, C), lambda b: (b, 0, 0)),
                   pl.BlockSpec((1, NCOARSE, C), lambda b: (b, 0, 0))],
        out_shape=[jax.ShapeDtypeStruct((B, NCOARSE, C), f32),
                   jax.ShapeDtypeStruct((B, NCOARSE, C), f32)],
    )(pq_t, W_q.T, b_q[None, :], pkv_t, W_key.T, b_key[None, :])

    tk = pl.pallas_call(
        _topk_body,
        grid=(B, SPLITM),
        in_specs=[pl.BlockSpec((1, 2 * HC, C), lambda b, i: (b, i, 0)),
                  pl.BlockSpec((1, NCOARSE, C), lambda b, i: (b, 0, 0)),
                  pl.BlockSpec((SPLITM, HC), lambda b, i: (0, 0))],
        out_specs=pl.BlockSpec((1, SPLITM, 2), lambda b, i: (b, i, 0)),
        out_shape=jax.ShapeDtypeStruct((B, NBLK, 2), jnp.int32),
    )(qd, kd, pair)

    qt4 = qt.reshape(B, NBLK, NQ, C)
    keyr4 = keyr.reshape(B, NCOARSE, TILE, C)
    valr4 = valr.reshape(B, NCOARSE, TILE, C)

    grid_spec = pltpu.PrefetchScalarGridSpec(
        num_scalar_prefetch=1,
        grid=(B, NBLK),
        in_specs=[
            pl.BlockSpec((1, 1, NQ, C), lambda b, j, tkr: (b, j, 0, 0)),
            pl.BlockSpec((1, 1, TILE, C),
                         lambda b, j, tkr: (b, tkr[(b * NBLK + j) * 2], 0, 0)),
            pl.BlockSpec((1, 1, TILE, C),
                         lambda b, j, tkr: (b, tkr[(b * NBLK + j) * 2 + 1], 0, 0)),
            pl.BlockSpec((1, 1, TILE, C),
                         lambda b, j, tkr: (b, tkr[(b * NBLK + j) * 2], 0, 0)),
            pl.BlockSpec((1, 1, TILE, C),
                         lambda b, j, tkr: (b, tkr[(b * NBLK + j) * 2 + 1], 0, 0)),
        ],
        out_specs=pl.BlockSpec((1, 1, NQ, C), lambda b, j, tkr: (b, j, 0, 0)),
    )
    att = pl.pallas_call(
        _phase2_body,
        grid_spec=grid_spec,
        out_shape=jax.ShapeDtypeStruct((B, NBLK, NQ, C), f32),
    )(tk.reshape(-1), qt4, keyr4, keyr4, valr4, valr4)

    att2 = att.reshape(B, NPIX, C)
    cb = 2304
    out_b = pl.pallas_call(
        _final_proj_body,
        grid=(B, NPIX // cb),
        in_specs=[pl.BlockSpec((1, cb, C), lambda b, j: (b, j, 0)),
                  pl.BlockSpec((1, cb, C), lambda b, j: (b, j, 0)),
                  pl.BlockSpec((C, C), lambda b, j: (0, 0)),
                  pl.BlockSpec((C, C), lambda b, j: (0, 0)),
                  pl.BlockSpec((C, 1), lambda b, j: (0, 0))],
        out_specs=pl.BlockSpec((1, C, cb), lambda b, j: (b, 0, j)),
        out_shape=jax.ShapeDtypeStruct((B, C, NPIX), f32),
    )(xq_r, att2, W_proj[:, 0:C], W_proj[:, C:2 * C], b_proj[:, None])

    out = (out_b.reshape(B, C, SPLITM, SPLITM, SFINE, SFINE)
           .transpose(0, 1, 2, 4, 3, 5).reshape(B, C, H, W))
    return out


# speedup vs baseline: 1.7999x; 1.3418x over previous
import jax
import jax.numpy as jnp
from jax import lax
from jax.experimental import pallas as pl
from jax.experimental.pallas import tpu as pltpu

B, C, H, W = 2, 96, 384, 384
NPIX = H * W
FAC = 6
HC = H // FAC
NCOARSE = HC * HC
SPLITM = 32
NBLK = SPLITM * SPLITM
SFINE = 12
NQ = SFINE * SFINE
TILE = FAC * FAC
G = 8
INV_SC2 = 1.0 / (96.0 ** 0.5)
PREC = lax.Precision.HIGHEST


def _dot(a, b):
    return jnp.dot(a, b, preferred_element_type=jnp.float32, precision=PREC)


def _dot_t(a, b):
    return lax.dot_general(a, b, (((1,), (1,)), ((), ())),
                           preferred_element_type=jnp.float32, precision=PREC)


def _mm_body(a_ref, b_ref, o_ref):
    o_ref[...] = _dot(a_ref[...], b_ref[...])


def _mm(a, bmat, rb):
    m, k = a.shape
    _, n = bmat.shape
    return pl.pallas_call(
        _mm_body,
        grid=(m // rb,),
        in_specs=[pl.BlockSpec((rb, k), lambda i: (i, 0)),
                  pl.BlockSpec((k, n), lambda i: (0, 0))],
        out_specs=pl.BlockSpec((rb, n), lambda i: (i, 0)),
        out_shape=jax.ShapeDtypeStruct((m, n), jnp.float32),
    )(a, bmat)


def _cproj_body(pq_ref, wqt_ref, bq_ref, pk_ref, wkt_ref, bk_ref,
                qd_ref, kd_ref):
    qd_ref[0] = _dot(pq_ref[0], wqt_ref[...]) + bq_ref[...]
    kd_ref[0] = _dot(pk_ref[0], wkt_ref[...]) + bk_ref[...]


def _topk_body(qd_ref, kd_ref, pair_ref, o_ref):
    q = qd_ref[0]
    k = kd_ref[0]
    e = (2.0 * _dot_t(q, k) - jnp.sum(k * k, axis=-1)[None, :]) * INV_SC2
    m = jnp.max(e, axis=-1, keepdims=True)
    p = jnp.exp(e - m)
    a = p / jnp.sum(p, axis=-1, keepdims=True)
    srow = a[0:HC, :] + a[HC:2 * HC, :]
    s = _dot(pair_ref[...], srow)
    idx = lax.broadcasted_iota(jnp.int32, s.shape, 1)
    m1 = jnp.max(s, axis=-1, keepdims=True)
    i1 = jnp.min(jnp.where(s == m1, idx, NCOARSE), axis=-1, keepdims=True)
    s2 = jnp.where(idx == i1, -jnp.inf, s)
    m2 = jnp.max(s2, axis=-1, keepdims=True)
    i2 = jnp.min(jnp.where(s2 == m2, idx, NCOARSE), axis=-1, keepdims=True)
    o_ref[0] = jnp.concatenate([i1, i2], axis=-1)


def _phase2_body(tk_ref, xq_ref, *rest):
    del tk_ref
    tiles = rest[:2 * G]
    wqt_ref, bq_ref, wkvt_ref, bkv_ref, o_ref = rest[2 * G:]
    wqt = wqt_ref[...]
    bq = bq_ref[...]
    wkvt = wkvt_ref[...]
    bkv = bkv_ref[...]
    for i in range(G):
        q = _dot(xq_ref[0, i * NQ:(i + 1) * NQ, :], wqt) + bq
        kv0 = _dot(tiles[2 * i][0, 0], wkvt) + bkv
        kv1 = _dot(tiles[2 * i + 1][0, 0], wkvt) + bkv
        k0, v0 = kv0[:, 0:C], kv0[:, C:2 * C]
        k1, v1 = kv1[:, 0:C], kv1[:, C:2 * C]
        e0 = (2.0 * _dot_t(q, k0) - jnp.sum(k0 * k0, axis=-1)[None, :]) * INV_SC2
        e1 = (2.0 * _dot_t(q, k1) - jnp.sum(k1 * k1, axis=-1)[None, :]) * INV_SC2
        m = jnp.maximum(jnp.max(e0, axis=-1, keepdims=True),
                        jnp.max(e1, axis=-1, keepdims=True))
        p0 = jnp.exp(e0 - m)
        p1 = jnp.exp(e1 - m)
        z = (jnp.sum(p0, axis=-1, keepdims=True)
             + jnp.sum(p1, axis=-1, keepdims=True))
        o_ref[0, i * NQ:(i + 1) * NQ, :] = _dot(p0 / z, v0) + _dot(p1 / z, v1)


def _final_proj_body(xq_ref, at_ref, wp1_ref, wp2_ref, bp_ref, o_ref):
    o_ref[0] = (lax.dot_general(wp1_ref[...], xq_ref[0], (((1,), (1,)), ((), ())),
                                preferred_element_type=jnp.float32, precision=PREC)
                + lax.dot_general(wp2_ref[...], at_ref[0], (((1,), (1,)), ((), ())),
                                  preferred_element_type=jnp.float32, precision=PREC)
                + bp_ref[...])


def _pool_img(x, pt):
    a = x.reshape(B * C * H, W)
    c1 = _mm(a, pt, 1024)
    c1 = c1.reshape(B, C, H, HC).transpose(0, 1, 3, 2).reshape(B * C * HC, H)
    c2 = _mm(c1, pt, 1024)
    return c2.reshape(B, C, HC, HC).transpose(0, 3, 2, 1).reshape(B, NCOARSE, C)


def kernel(xKeyValue, xQuery, W_val, b_val, W_key, b_key, W_q, b_q,
           W_proj, b_proj):
    f32 = jnp.float32
    pt = (jnp.arange(W)[:, None] // FAC == jnp.arange(HC)[None, :]).astype(f32) / FAC
    pair = (jnp.arange(SPLITM)[:, None] == jnp.arange(HC)[None, :] // 2).astype(f32)

    xkv_r = (xKeyValue.reshape(B, C, HC, FAC, HC, FAC)
             .transpose(0, 2, 4, 3, 5, 1).reshape(B, NCOARSE, TILE, C))
    xq_r = (xQuery.reshape(B, C, SPLITM, SFINE, SPLITM, SFINE)
            .transpose(0, 2, 4, 3, 5, 1).reshape(B, NPIX, C))

    pq_t = _pool_img(xQuery, pt)
    pkv_t = _pool_img(xKeyValue, pt)

    qd, kd = pl.pallas_call(
        _cproj_body,
        grid=(B,),
        in_specs=[pl.BlockSpec((1, NCOARSE, C), lambda b: (b, 0, 0)),
                  pl.BlockSpec((C, C), lambda b: (0, 0)),
                  pl.BlockSpec((1, C), lambda b: (0, 0)),
                  pl.BlockSpec((1, NCOARSE, C), lambda b: (b, 0, 0)),
                  pl.BlockSpec((C, C), lambda b: (0, 0)),
                  pl.BlockSpec((1, C), lambda b: (0, 0))],
        out_specs=[pl.BlockSpec((1, NCOARSE, C), lambda b: (b, 0, 0)),
                   pl.BlockSpec((1, NCOARSE, C), lambda b: (b, 0, 0))],
        out_shape=[jax.ShapeDtypeStruct((B, NCOARSE, C), f32),
                   jax.ShapeDtypeStruct((B, NCOARSE, C), f32)],
    )(pq_t, W_q.T, b_q[None, :], pkv_t, W_key.T, b_key[None, :])

    tk = pl.pallas_call(
        _topk_body,
        grid=(B, SPLITM),
        in_specs=[pl.BlockSpec((1, 2 * HC, C), lambda b, i: (b, i, 0)),
                  pl.BlockSpec((1, NCOARSE, C), lambda b, i: (b, 0, 0)),
                  pl.BlockSpec((SPLITM, HC), lambda b, i: (0, 0))],
        out_specs=pl.BlockSpec((1, SPLITM, 2), lambda b, i: (b, i, 0)),
        out_shape=jax.ShapeDtypeStruct((B, NBLK, 2), jnp.int32),
    )(qd, kd, pair)

    def _tile_spec(i, kap):
        def imap(b, j, tkr, i=i, kap=kap):
            return (b, tkr[(b * NBLK + G * j + i) * 2 + kap], 0, 0)
        return pl.BlockSpec((1, 1, TILE, C), imap)

    tile_specs = []
    for i in range(G):
        tile_specs.append(_tile_spec(i, 0))
        tile_specs.append(_tile_spec(i, 1))

    grid_spec = pltpu.PrefetchScalarGridSpec(
        num_scalar_prefetch=1,
        grid=(B, NBLK // G),
        in_specs=[pl.BlockSpec((1, G * NQ, C), lambda b, j, tkr: (b, j, 0))]
                 + tile_specs
                 + [pl.BlockSpec((C, C), lambda b, j, tkr: (0, 0)),
                    pl.BlockSpec((1, C), lambda b, j, tkr: (0, 0)),
                    pl.BlockSpec((C, 2 * C), lambda b, j, tkr: (0, 0)),
                    pl.BlockSpec((1, 2 * C), lambda b, j, tkr: (0, 0))],
        out_specs=pl.BlockSpec((1, G * NQ, C), lambda b, j, tkr: (b, j, 0)),
    )
    att = pl.pallas_call(
        _phase2_body,
        grid_spec=grid_spec,
        out_shape=jax.ShapeDtypeStruct((B, NPIX, C), f32),
    )(tk.reshape(-1), xq_r, *([xkv_r] * (2 * G)),
      W_q.T, b_q[None, :],
      jnp.concatenate([W_key.T, W_val.T], axis=1),
      jnp.concatenate([b_key, b_val])[None, :])

    cb = 2304
    out_b = pl.pallas_call(
        _final_proj_body,
        grid=(B, NPIX // cb),
        in_specs=[pl.BlockSpec((1, cb, C), lambda b, j: (b, j, 0)),
                  pl.BlockSpec((1, cb, C), lambda b, j: (b, j, 0)),
                  pl.BlockSpec((C, C), lambda b, j: (0, 0)),
                  pl.BlockSpec((C, C), lambda b, j: (0, 0)),
                  pl.BlockSpec((C, 1), lambda b, j: (0, 0))],
        out_specs=pl.BlockSpec((1, C, cb), lambda b, j: (b, 0, j)),
        out_shape=jax.ShapeDtypeStruct((B, C, NPIX), f32),
    )(xq_r, att, W_proj[:, 0:C], W_proj[:, C:2 * C], b_proj[:, None])

    out = (out_b.reshape(B, C, SPLITM, SPLITM, SFINE, SFINE)
           .transpose(0, 1, 2, 4, 3, 5).reshape(B, C, H, W))
    return out
